# Initial kernel scaffold; baseline (speedup 1.0000x reference)
import jax
import jax.numpy as jnp
from jax.experimental import pallas as pl


def kernel(x, expert_weights, expert_indices, W1, W2, W3, g_in, b_in, g_out, b_out, g_ln, b_ln):
    raise NotImplementedError("write your pallas kernel here")


# speedup vs baseline: 1.2832x; 1.2832x over previous
import jax
import jax.numpy as jnp
from jax.experimental import pallas as pl
from jax.experimental.pallas import tpu as pltpu

TILE = 128
EPS = 1e-5


def _ln(v, g, b):
    mu = jnp.mean(v, axis=1, keepdims=True)
    var = jnp.mean(jnp.square(v - mu), axis=1, keepdims=True)
    return (v - mu) * jax.lax.rsqrt(var + EPS) * g + b


def _ffn_kernel(ut_ref, ue_ref, uf_ref, es_ref, ee_ref, tid_ref,
                x_ref, w1_ref, w2_ref, w3_ref, lnp_ref,
                out_ref, xs_ref, sem):
    g = pl.program_id(0)
    e = ue_ref[g]
    first = uf_ref[g]
    base = ut_ref[g] * TILE

    def gather_start(i, c):
        tid = tid_ref[base + i]
        pltpu.make_async_copy(x_ref.at[pl.ds(tid, 1), :],
                              xs_ref.at[pl.ds(i, 1), :], sem).start()
        return c
    jax.lax.fori_loop(0, TILE, gather_start, 0)

    def gather_wait(i, c):
        pltpu.make_async_copy(x_ref.at[pl.ds(0, 1), :],
                              xs_ref.at[pl.ds(0, 1), :], sem).wait()
        return c
    jax.lax.fori_loop(0, TILE, gather_wait, 0)

    xs = xs_ref[:, :]
    xn = _ln(xs, lnp_ref[0, 0, :][None, :], lnp_ref[0, 1, :][None, :])
    xnb = xn.astype(jnp.bfloat16)

    dims = (((1,), (1,)), ((), ()))
    rem = e % 3
    dff = w1_ref.shape[1]
    d = w1_ref.shape[2]
    nc = dff // d

    def ffn(use_w2):
        y = jnp.zeros((xs.shape[0], d), jnp.float32)
        for c in range(nc):
            lo = c * d
            w1c = w1_ref[0, lo:lo + d, :]
            h1 = jax.lax.dot_general(xnb, w1c, dims,
                                     preferred_element_type=jnp.float32)
            if use_w2:
                w2c = w2_ref[0, lo:lo + d, :]
                h2 = jax.lax.dot_general(xnb, w2c, dims,
                                         preferred_element_type=jnp.float32)
                h = (h2 * jax.lax.logistic(h2)) * h2 * h1
            else:
                gelu = 0.5 * h1 * (1.0 + jax.lax.erf(h1 * 0.7071067811865475))
                h = jnp.where(rem == 1, gelu, jnp.maximum(h1, 0.0))
            w3c = w3_ref[0, :, lo:lo + d]
            y = y + jax.lax.dot_general(h.astype(jnp.bfloat16), w3c, dims,
                                        preferred_element_type=jnp.float32)
        return y

    y = jax.lax.cond(rem == 0, lambda: ffn(True), lambda: ffn(False))
    eo = _ln(xs + y, lnp_ref[0, 2, :][None, :], lnp_ref[0, 3, :][None, :])

    r = jax.lax.broadcasted_iota(jnp.int32, (xs.shape[0], 1), 0) + base
    mask = (r >= es_ref[e]) & (r < ee_ref[e])
    prev = jnp.where(first == 1, jnp.zeros_like(eo), out_ref[:, :])
    out_ref[:, :] = jnp.where(mask, eo, prev)


def _combine_kernel(inv_ref, os_ref, l0_ref, l1_ref, gln_ref, bln_ref,
                    out_ref, g0_ref, g1_ref):
    t = pl.program_id(0)
    base = t * TILE

    def body(i, c):
        p = 2 * (base + i)
        r0 = inv_ref[p]
        r1 = inv_ref[p + 1]
        g0_ref[pl.ds(i, 1), :] = os_ref[pl.ds(r0, 1), :]
        g1_ref[pl.ds(i, 1), :] = os_ref[pl.ds(r1, 1), :]
        return c
    jax.lax.fori_loop(0, TILE, body, 0)

    d0 = l0_ref[:, :] - l1_ref[:, :]
    w0 = jax.lax.logistic(d0)
    w1 = jax.lax.logistic(-d0)
    c = w0 * g0_ref[:, :] + w1 * g1_ref[:, :]
    out_ref[:, :] = _ln(c, gln_ref[0, :][None, :], bln_ref[0, :][None, :])


def kernel(x, expert_weights, expert_indices, W1, W2, W3,
           g_in, b_in, g_out, b_out, g_ln, b_ln):
    b, s, d = x.shape
    k = expert_indices.shape[-1]
    n_tok = b * s
    n = n_tok * k
    e_num, dff, _ = W1.shape
    nt = n // TILE
    nu = nt + e_num - 1

    xf = x.reshape(n_tok, d)

    eid_flat = expert_indices.reshape(n).astype(jnp.int32)
    order = jnp.argsort(eid_flat).astype(jnp.int32)
    sorted_eid = jnp.take(eid_flat, order)
    sorted_tid = (order // k).astype(jnp.int32)
    inv_order = jnp.argsort(order).astype(jnp.int32)

    counts = jnp.bincount(eid_flat, length=e_num).astype(jnp.int32)
    ends = jnp.cumsum(counts)
    starts = ends - counts
    t0 = starts // TILE
    t1 = jnp.where(counts > 0, (ends - 1) // TILE + 1, t0)
    ntiles_e = (t1 - t0).astype(jnp.int32)
    ucum = jnp.cumsum(ntiles_e)
    ustart = ucum - ntiles_e
    total_units = ucum[-1]

    sarr = jnp.arange(nu, dtype=jnp.int32)
    e_of_s = jnp.clip(jnp.searchsorted(ucum, sarr, side='right'),
                      0, e_num - 1).astype(jnp.int32)
    tile_raw = jnp.take(t0, e_of_s) + (sarr - jnp.take(ustart, e_of_s))
    valid = sarr < total_units
    unit_eid = jnp.where(valid, e_of_s, sorted_eid[-1]).astype(jnp.int32)
    unit_tile = jnp.where(valid, jnp.clip(tile_raw, 0, nt - 1),
                          nt - 1).astype(jnp.int32)
    unit_first = jnp.concatenate(
        [jnp.ones((1,), jnp.int32),
         (unit_tile[1:] != unit_tile[:-1]).astype(jnp.int32)])

    W1b = W1.astype(jnp.bfloat16)
    W2b = W2.astype(jnp.bfloat16)
    W3b = W3.astype(jnp.bfloat16)

    grid1 = pltpu.PrefetchScalarGridSpec(
        num_scalar_prefetch=6,
        grid=(nu,),
        in_specs=[
            pl.BlockSpec(memory_space=pltpu.MemorySpace.HBM),
            pl.BlockSpec((1, dff, d), lambda g, ut, ue, *_: (ue[g], 0, 0)),
            pl.BlockSpec((1, dff, d),
                         lambda g, ut, ue, *_: (3 * (ue[g] // 3), 0, 0)),
            pl.BlockSpec((1, d, dff), lambda g, ut, ue, *_: (ue[g], 0, 0)),
            pl.BlockSpec((1, 4, d), lambda g, ut, ue, *_: (ue[g], 0, 0)),
        ],
        out_specs=pl.BlockSpec((TILE, d), lambda g, ut, ue, *_: (ut[g], 0)),
        scratch_shapes=[pltpu.VMEM((TILE, d), jnp.float32),
                        pltpu.SemaphoreType.DMA],
    )
    out_sorted = pl.pallas_call(
        _ffn_kernel,
        grid_spec=grid1,
        out_shape=jax.ShapeDtypeStruct((n, d), jnp.float32),
        compiler_params=pltpu.CompilerParams(
            dimension_semantics=("arbitrary",)),
    )(unit_tile, unit_eid, unit_first, starts.astype(jnp.int32),
      ends.astype(jnp.int32), sorted_tid,
      xf, W1b, W2b, W3b,
      jnp.stack([g_in, b_in, g_out, b_out], axis=1))

    ntt = n_tok // TILE
    ew = expert_weights.reshape(n_tok, k)
    l0 = ew[:, 0].reshape(n_tok, 1)
    l1 = ew[:, 1].reshape(n_tok, 1)
    gln = g_ln.reshape(1, d)
    bln = b_ln.reshape(1, d)

    grid2 = pltpu.PrefetchScalarGridSpec(
        num_scalar_prefetch=1,
        grid=(ntt,),
        in_specs=[
            pl.BlockSpec((n, d), lambda t, inv: (0, 0)),
            pl.BlockSpec((TILE, 1), lambda t, inv: (t, 0)),
            pl.BlockSpec((TILE, 1), lambda t, inv: (t, 0)),
            pl.BlockSpec((1, d), lambda t, inv: (0, 0)),
            pl.BlockSpec((1, d), lambda t, inv: (0, 0)),
        ],
        out_specs=pl.BlockSpec((TILE, d), lambda t, inv: (t, 0)),
        scratch_shapes=[pltpu.VMEM((TILE, d), jnp.float32),
                        pltpu.VMEM((TILE, d), jnp.float32)],
    )
    combined = pl.pallas_call(
        _combine_kernel,
        grid_spec=grid2,
        out_shape=jax.ShapeDtypeStruct((n_tok, d), jnp.float32),
        compiler_params=pltpu.CompilerParams(
            dimension_semantics=("arbitrary",)),
    )(inv_order, out_sorted, l0, l1, gln, bln)

    return combined.reshape(b, s, d)


# speedup vs baseline: 1.3555x; 1.0563x over previous
import jax
import jax.numpy as jnp
from jax.experimental import pallas as pl
from jax.experimental.pallas import tpu as pltpu

TILE = 128
EPS = 1e-5


def _ln(v, g, b):
    mu = jnp.mean(v, axis=1, keepdims=True)
    var = jnp.mean(jnp.square(v - mu), axis=1, keepdims=True)
    return (v - mu) * jax.lax.rsqrt(var + EPS) * g + b


def _ffn_kernel(ut_ref, ue_ref, uf_ref, es_ref, ee_ref, tid_ref,
                x_ref, w1_ref, w2_ref, w3_ref, lnp_ref,
                out_ref, xs_ref, sem):
    g = pl.program_id(0)
    e = ue_ref[g]
    first = uf_ref[g]
    t = ut_ref[g]
    b = jax.lax.rem(t, 2)
    base = t * TILE

    def issue(tile_idx, buf):
        bb = tile_idx * TILE

        def body(i, c):
            tid = tid_ref[bb + i]
            pltpu.make_async_copy(x_ref.at[pl.ds(tid, 1), :],
                                  xs_ref.at[buf, pl.ds(i, 1), :],
                                  sem.at[buf]).start()
            return c
        jax.lax.fori_loop(0, TILE, body, 0)

    pl.when(g == 0)(lambda: issue(t, b))

    def do_wait():
        def body(i, c):
            pltpu.make_async_copy(x_ref.at[pl.ds(0, 1), :],
                                  xs_ref.at[0, pl.ds(0, 1), :],
                                  sem.at[b]).wait()
            return c
        jax.lax.fori_loop(0, TILE, body, 0)
    pl.when(first == 1)(do_wait)

    pl.when(uf_ref[g + 1] == 1)(lambda: issue(t + 1, 1 - b))

    xs = xs_ref[b]
    xn = _ln(xs, lnp_ref[0, 0, :][None, :], lnp_ref[0, 1, :][None, :])
    xnb = xn.astype(jnp.bfloat16)

    dims = (((1,), (1,)), ((), ()))
    rem = e % 3
    dff = w1_ref.shape[1]
    d = w1_ref.shape[2]
    nc = dff // d

    def ffn(use_w2):
        y = jnp.zeros((xs.shape[0], d), jnp.float32)
        for c in range(nc):
            lo = c * d
            w1c = w1_ref[0, lo:lo + d, :]
            h1 = jax.lax.dot_general(xnb, w1c, dims,
                                     preferred_element_type=jnp.float32)
            if use_w2:
                w2c = w2_ref[0, lo:lo + d, :]
                h2 = jax.lax.dot_general(xnb, w2c, dims,
                                         preferred_element_type=jnp.float32)
                h = (h2 * jax.lax.logistic(h2)) * h2 * h1
            else:
                gelu = 0.5 * h1 * (1.0 + jax.lax.erf(h1 * 0.7071067811865475))
                h = jnp.where(rem == 1, gelu, jnp.maximum(h1, 0.0))
            w3c = w3_ref[0, :, lo:lo + d]
            y = y + jax.lax.dot_general(h.astype(jnp.bfloat16), w3c, dims,
                                        preferred_element_type=jnp.float32)
        return y

    y = jax.lax.cond(rem == 0, lambda: ffn(True), lambda: ffn(False))
    eo = _ln(xs + y, lnp_ref[0, 2, :][None, :], lnp_ref[0, 3, :][None, :])

    r = jax.lax.broadcasted_iota(jnp.int32, (xs.shape[0], 1), 0) + base
    mask = (r >= es_ref[e]) & (r < ee_ref[e])
    prev = jnp.where(first == 1, jnp.zeros_like(eo), out_ref[:, :])
    out_ref[:, :] = jnp.where(mask, eo, prev)


def _combine_kernel(inv_ref, os_ref, l0_ref, l1_ref, gln_ref, bln_ref,
                    out_ref, g0_ref, g1_ref):
    t = pl.program_id(0)
    base = t * TILE

    def body(i, c):
        p = 2 * (base + i)
        r0 = inv_ref[p]
        r1 = inv_ref[p + 1]
        g0_ref[pl.ds(i, 1), :] = os_ref[pl.ds(r0, 1), :]
        g1_ref[pl.ds(i, 1), :] = os_ref[pl.ds(r1, 1), :]
        return c
    jax.lax.fori_loop(0, TILE, body, 0)

    d0 = l0_ref[:, :] - l1_ref[:, :]
    w0 = jax.lax.logistic(d0)
    w1 = jax.lax.logistic(-d0)
    c = w0 * g0_ref[:, :] + w1 * g1_ref[:, :]
    out_ref[:, :] = _ln(c, gln_ref[0, :][None, :], bln_ref[0, :][None, :])


def kernel(x, expert_weights, expert_indices, W1, W2, W3,
           g_in, b_in, g_out, b_out, g_ln, b_ln):
    b, s, d = x.shape
    k = expert_indices.shape[-1]
    n_tok = b * s
    n = n_tok * k
    e_num, dff, _ = W1.shape
    nt = n // TILE
    nu = nt + e_num - 1

    xf = x.reshape(n_tok, d)

    eid_flat = expert_indices.reshape(n).astype(jnp.int32)
    order = jnp.argsort(eid_flat).astype(jnp.int32)
    sorted_eid = jnp.take(eid_flat, order)
    sorted_tid = (order // k).astype(jnp.int32)
    inv_order = jnp.argsort(order).astype(jnp.int32)

    counts = jnp.bincount(eid_flat, length=e_num).astype(jnp.int32)
    ends = jnp.cumsum(counts)
    starts = ends - counts
    t0 = starts // TILE
    t1 = jnp.where(counts > 0, (ends - 1) // TILE + 1, t0)
    ntiles_e = (t1 - t0).astype(jnp.int32)
    ucum = jnp.cumsum(ntiles_e)
    ustart = ucum - ntiles_e
    total_units = ucum[-1]

    sarr = jnp.arange(nu, dtype=jnp.int32)
    e_of_s = jnp.clip(jnp.searchsorted(ucum, sarr, side='right'),
                      0, e_num - 1).astype(jnp.int32)
    tile_raw = jnp.take(t0, e_of_s) + (sarr - jnp.take(ustart, e_of_s))
    valid = sarr < total_units
    unit_eid = jnp.where(valid, e_of_s, sorted_eid[-1]).astype(jnp.int32)
    unit_tile = jnp.where(valid, jnp.clip(tile_raw, 0, nt - 1),
                          nt - 1).astype(jnp.int32)
    unit_first = jnp.concatenate(
        [jnp.ones((1,), jnp.int32),
         (unit_tile[1:] != unit_tile[:-1]).astype(jnp.int32),
         jnp.zeros((1,), jnp.int32)])

    W1b = W1.astype(jnp.bfloat16)
    W2b = W2.astype(jnp.bfloat16)
    W3b = W3.astype(jnp.bfloat16)

    grid1 = pltpu.PrefetchScalarGridSpec(
        num_scalar_prefetch=6,
        grid=(nu,),
        in_specs=[
            pl.BlockSpec(memory_space=pltpu.MemorySpace.HBM),
            pl.BlockSpec((1, dff, d), lambda g, ut, ue, *_: (ue[g], 0, 0)),
            pl.BlockSpec((1, dff, d),
                         lambda g, ut, ue, *_: (3 * (ue[g] // 3), 0, 0)),
            pl.BlockSpec((1, d, dff), lambda g, ut, ue, *_: (ue[g], 0, 0)),
            pl.BlockSpec((1, 4, d), lambda g, ut, ue, *_: (ue[g], 0, 0)),
        ],
        out_specs=pl.BlockSpec((TILE, d), lambda g, ut, ue, *_: (ut[g], 0)),
        scratch_shapes=[pltpu.VMEM((2, TILE, d), jnp.float32),
                        pltpu.SemaphoreType.DMA((2,))],
    )
    out_sorted = pl.pallas_call(
        _ffn_kernel,
        grid_spec=grid1,
        out_shape=jax.ShapeDtypeStruct((n, d), jnp.float32),
        compiler_params=pltpu.CompilerParams(
            dimension_semantics=("arbitrary",)),
    )(unit_tile, unit_eid, unit_first, starts.astype(jnp.int32),
      ends.astype(jnp.int32), sorted_tid,
      xf, W1b, W2b, W3b,
      jnp.stack([g_in, b_in, g_out, b_out], axis=1))

    ntt = n_tok // TILE
    ew = expert_weights.reshape(n_tok, k)
    l0 = ew[:, 0].reshape(n_tok, 1)
    l1 = ew[:, 1].reshape(n_tok, 1)
    gln = g_ln.reshape(1, d)
    bln = b_ln.reshape(1, d)

    grid2 = pltpu.PrefetchScalarGridSpec(
        num_scalar_prefetch=1,
        grid=(ntt,),
        in_specs=[
            pl.BlockSpec((n, d), lambda t, inv: (0, 0)),
            pl.BlockSpec((TILE, 1), lambda t, inv: (t, 0)),
            pl.BlockSpec((TILE, 1), lambda t, inv: (t, 0)),
            pl.BlockSpec((1, d), lambda t, inv: (0, 0)),
            pl.BlockSpec((1, d), lambda t, inv: (0, 0)),
        ],
        out_specs=pl.BlockSpec((TILE, d), lambda t, inv: (t, 0)),
        scratch_shapes=[pltpu.VMEM((TILE, d), jnp.float32),
                        pltpu.VMEM((TILE, d), jnp.float32)],
    )
    combined = pl.pallas_call(
        _combine_kernel,
        grid_spec=grid2,
        out_shape=jax.ShapeDtypeStruct((n_tok, d), jnp.float32),
        compiler_params=pltpu.CompilerParams(
            dimension_semantics=("arbitrary",)),
    )(inv_order, out_sorted, l0, l1, gln, bln)

    return combined.reshape(b, s, d)


# speedup vs baseline: 1.3589x; 1.0025x over previous
import jax
import jax.numpy as jnp
from jax.experimental import pallas as pl
from jax.experimental.pallas import tpu as pltpu

TILE = 128
EPS = 1e-5


def _ln(v, g, b):
    mu = jnp.mean(v, axis=1, keepdims=True)
    var = jnp.mean(jnp.square(v - mu), axis=1, keepdims=True)
    return (v - mu) * jax.lax.rsqrt(var + EPS) * g + b


def _ffn_kernel(ut_ref, ue_ref, uf_ref, es_ref, ee_ref, tid_ref,
                x_ref, w1_ref, w2_ref, w3_ref, lnp_ref,
                out_ref, xs_ref, sem):
    g = pl.program_id(0)
    e = ue_ref[g]
    first = uf_ref[g]
    t = ut_ref[g]
    b = jax.lax.rem(t, 2)
    base = t * TILE

    def issue(tile_idx, buf):
        bb = tile_idx * TILE

        def body(i, c):
            tid = tid_ref[bb + i]
            pltpu.make_async_copy(x_ref.at[pl.ds(tid, 1), :],
                                  xs_ref.at[buf, pl.ds(i, 1), :],
                                  sem.at[buf]).start()
            return c
        jax.lax.fori_loop(0, TILE, body, 0)

    pl.when(g == 0)(lambda: issue(t, b))

    def do_wait():
        def body(i, c):
            pltpu.make_async_copy(x_ref.at[pl.ds(0, 1), :],
                                  xs_ref.at[0, pl.ds(0, 1), :],
                                  sem.at[b]).wait()
            return c
        jax.lax.fori_loop(0, TILE, body, 0)
    pl.when(first == 1)(do_wait)

    pl.when(uf_ref[g + 1] == 1)(lambda: issue(t + 1, 1 - b))

    xs = xs_ref[b]
    xn = _ln(xs, lnp_ref[0, 0, :][None, :], lnp_ref[0, 1, :][None, :])
    xnb = xn.astype(jnp.bfloat16)

    dims = (((1,), (1,)), ((), ()))
    rem = e % 3
    dff = w1_ref.shape[1]
    d = w1_ref.shape[2]
    cs = 2 * d
    nc = dff // cs

    def ffn(use_w2):
        y = jnp.zeros((xs.shape[0], d), jnp.float32)
        for c in range(nc):
            lo = c * cs
            w1c = w1_ref[0, lo:lo + cs, :]
            h1 = jax.lax.dot_general(xnb, w1c, dims,
                                     preferred_element_type=jnp.float32)
            if use_w2:
                w2c = w2_ref[0, lo:lo + cs, :]
                h2 = jax.lax.dot_general(xnb, w2c, dims,
                                         preferred_element_type=jnp.float32)
                h = (h2 * jax.lax.logistic(h2)) * h2 * h1
            else:
                gelu = 0.5 * h1 * (1.0 + jax.lax.erf(h1 * 0.7071067811865475))
                h = jnp.where(rem == 1, gelu, jnp.maximum(h1, 0.0))
            w3c = w3_ref[0, :, lo:lo + cs]
            y = y + jax.lax.dot_general(h.astype(jnp.bfloat16), w3c, dims,
                                        preferred_element_type=jnp.float32)
        return y

    y = jax.lax.cond(rem == 0, lambda: ffn(True), lambda: ffn(False))
    eo = _ln(xs + y, lnp_ref[0, 2, :][None, :], lnp_ref[0, 3, :][None, :])

    r = jax.lax.broadcasted_iota(jnp.int32, (xs.shape[0], 1), 0) + base
    mask = (r >= es_ref[e]) & (r < ee_ref[e])
    prev = jnp.where(first == 1, jnp.zeros_like(eo), out_ref[:, :])
    out_ref[:, :] = jnp.where(mask, eo, prev)


def _combine_kernel(inv_ref, os_ref, l0_ref, l1_ref, gln_ref, bln_ref,
                    out_ref, g0_ref, g1_ref):
    t = pl.program_id(0)
    base = t * TILE

    def body(i, c):
        p = 2 * (base + i)
        r0 = inv_ref[p]
        r1 = inv_ref[p + 1]
        g0_ref[pl.ds(i, 1), :] = os_ref[pl.ds(r0, 1), :]
        g1_ref[pl.ds(i, 1), :] = os_ref[pl.ds(r1, 1), :]
        return c
    jax.lax.fori_loop(0, TILE, body, 0)

    d0 = l0_ref[:, :] - l1_ref[:, :]
    w0 = jax.lax.logistic(d0)
    w1 = jax.lax.logistic(-d0)
    c = w0 * g0_ref[:, :] + w1 * g1_ref[:, :]
    out_ref[:, :] = _ln(c, gln_ref[0, :][None, :], bln_ref[0, :][None, :])


def kernel(x, expert_weights, expert_indices, W1, W2, W3,
           g_in, b_in, g_out, b_out, g_ln, b_ln):
    b, s, d = x.shape
    k = expert_indices.shape[-1]
    n_tok = b * s
    n = n_tok * k
    e_num, dff, _ = W1.shape
    nt = n // TILE
    nu = nt + e_num - 1

    xf = x.reshape(n_tok, d)

    eid_flat = expert_indices.reshape(n).astype(jnp.int32)
    order = jnp.argsort(eid_flat).astype(jnp.int32)
    sorted_eid = jnp.take(eid_flat, order)
    sorted_tid = (order // k).astype(jnp.int32)
    inv_order = jnp.argsort(order).astype(jnp.int32)

    counts = jnp.bincount(eid_flat, length=e_num).astype(jnp.int32)
    ends = jnp.cumsum(counts)
    starts = ends - counts
    t0 = starts // TILE
    t1 = jnp.where(counts > 0, (ends - 1) // TILE + 1, t0)
    ntiles_e = (t1 - t0).astype(jnp.int32)
    ucum = jnp.cumsum(ntiles_e)
    ustart = ucum - ntiles_e
    total_units = ucum[-1]

    sarr = jnp.arange(nu, dtype=jnp.int32)
    e_of_s = jnp.clip(jnp.searchsorted(ucum, sarr, side='right'),
                      0, e_num - 1).astype(jnp.int32)
    tile_raw = jnp.take(t0, e_of_s) + (sarr - jnp.take(ustart, e_of_s))
    valid = sarr < total_units
    unit_eid = jnp.where(valid, e_of_s, sorted_eid[-1]).astype(jnp.int32)
    unit_tile = jnp.where(valid, jnp.clip(tile_raw, 0, nt - 1),
                          nt - 1).astype(jnp.int32)
    unit_first = jnp.concatenate(
        [jnp.ones((1,), jnp.int32),
         (unit_tile[1:] != unit_tile[:-1]).astype(jnp.int32),
         jnp.zeros((1,), jnp.int32)])

    W1b = W1.astype(jnp.bfloat16)
    W2b = W2.astype(jnp.bfloat16)
    W3b = W3.astype(jnp.bfloat16)

    grid1 = pltpu.PrefetchScalarGridSpec(
        num_scalar_prefetch=6,
        grid=(nu,),
        in_specs=[
            pl.BlockSpec(memory_space=pltpu.MemorySpace.HBM),
            pl.BlockSpec((1, dff, d), lambda g, ut, ue, *_: (ue[g], 0, 0)),
            pl.BlockSpec((1, dff, d),
                         lambda g, ut, ue, *_: (3 * (ue[g] // 3), 0, 0)),
            pl.BlockSpec((1, d, dff), lambda g, ut, ue, *_: (ue[g], 0, 0)),
            pl.BlockSpec((1, 4, d), lambda g, ut, ue, *_: (ue[g], 0, 0)),
        ],
        out_specs=pl.BlockSpec((TILE, d), lambda g, ut, ue, *_: (ut[g], 0)),
        scratch_shapes=[pltpu.VMEM((2, TILE, d), jnp.float32),
                        pltpu.SemaphoreType.DMA((2,))],
    )
    out_sorted = pl.pallas_call(
        _ffn_kernel,
        grid_spec=grid1,
        out_shape=jax.ShapeDtypeStruct((n, d), jnp.float32),
        compiler_params=pltpu.CompilerParams(
            dimension_semantics=("arbitrary",)),
    )(unit_tile, unit_eid, unit_first, starts.astype(jnp.int32),
      ends.astype(jnp.int32), sorted_tid,
      xf, W1b, W2b, W3b,
      jnp.stack([g_in, b_in, g_out, b_out], axis=1))

    ntt = n_tok // TILE
    ew = expert_weights.reshape(n_tok, k)
    l0 = ew[:, 0].reshape(n_tok, 1)
    l1 = ew[:, 1].reshape(n_tok, 1)
    gln = g_ln.reshape(1, d)
    bln = b_ln.reshape(1, d)

    grid2 = pltpu.PrefetchScalarGridSpec(
        num_scalar_prefetch=1,
        grid=(ntt,),
        in_specs=[
            pl.BlockSpec((n, d), lambda t, inv: (0, 0)),
            pl.BlockSpec((TILE, 1), lambda t, inv: (t, 0)),
            pl.BlockSpec((TILE, 1), lambda t, inv: (t, 0)),
            pl.BlockSpec((1, d), lambda t, inv: (0, 0)),
            pl.BlockSpec((1, d), lambda t, inv: (0, 0)),
        ],
        out_specs=pl.BlockSpec((TILE, d), lambda t, inv: (t, 0)),
        scratch_shapes=[pltpu.VMEM((TILE, d), jnp.float32),
                        pltpu.VMEM((TILE, d), jnp.float32)],
    )
    combined = pl.pallas_call(
        _combine_kernel,
        grid_spec=grid2,
        out_shape=jax.ShapeDtypeStruct((n_tok, d), jnp.float32),
        compiler_params=pltpu.CompilerParams(
            dimension_semantics=("arbitrary",)),
    )(inv_order, out_sorted, l0, l1, gln, bln)

    return combined.reshape(b, s, d)
